# baseline (speedup 1.0000x reference)
import functools

import jax
import jax.numpy as jnp
from jax import lax
from jax.experimental import pallas as pl
from jax.experimental.pallas import tpu as pltpu
from jax.experimental.pallas import tpu_sc as plsc

_NC, _NS = 2, 16
_NW = _NC * _NS


def _vtab_body(v_ref, ws_ref, wr_ref, b_ref, o_ref):
    wsr = ws_ref[...] + wr_ref[...]
    o_ref[...] = (
        jnp.dot(v_ref[...], wsr, preferred_element_type=jnp.float32) + b_ref[...]
    )


def _combine_body(g_ref, e_ref, we_ref, o_ref):
    o_ref[...] = g_ref[...] + jnp.dot(
        e_ref[...].astype(jnp.bfloat16),
        we_ref[...].astype(jnp.bfloat16),
        preferred_element_type=jnp.float32,
    )


def _combine_alias_body(prev_ref, g_ref, e_ref, we_ref, o_ref):
    del prev_ref
    _combine_body(g_ref, e_ref, we_ref, o_ref)


def _make_gather(rows_k, D, roff):
    C = 128
    RW = rows_k // _NW
    nch = RW
    NBUF = 4
    LOOK = 2
    assert rows_k % _NW == 0 and RW % 8 == 0 and nch % NBUF == 0 and nch >= NBUF
    mesh = plsc.VectorSubcoreMesh(core_axis_name="c", subcore_axis_name="s")

    scratch = [pltpu.VMEM((RW, C), jnp.int32)]
    scratch += [pltpu.VMEM((C, D), jnp.float32) for _ in range(NBUF)]
    scratch += [pltpu.SemaphoreType.DMA for _ in range(2 * NBUF)]

    @functools.partial(
        pl.kernel,
        out_type=jax.ShapeDtypeStruct((rows_k * C, D), jnp.float32),
        mesh=mesh,
        scratch_types=scratch,
    )
    def gather(vtab_hbm, idx_hbm, out_hbm, idx_v, *bufs_sems):
        rows = bufs_sems[:NBUF]
        gsem = bufs_sems[NBUF : 2 * NBUF]
        ssem = bufs_sems[2 * NBUF :]
        wid = lax.axis_index("s") * _NC + lax.axis_index("c")
        base_r = roff + RW * wid
        pltpu.sync_copy(idx_hbm.at[pl.ds(base_r, RW)], idx_v)

        obase = RW * wid * C

        def issue_gather(j, b):
            pltpu.async_copy(vtab_hbm.at[idx_v.at[j]], rows[b], gsem[b])

        def wait_gather(j, b):
            pltpu.make_async_copy(vtab_hbm.at[idx_v.at[j]], rows[b], gsem[b]).wait()

        def wait_store(b):
            pltpu.make_async_copy(rows[b], out_hbm.at[pl.ds(obase, C)], ssem[b]).wait()

        for p in range(LOOK):
            issue_gather(p, p)

        @pl.loop(0, nch, step=NBUF)
        def _(o):
            for b in range(NBUF):
                g = o + b
                wait_gather(g, b)
                pltpu.async_copy(rows[b], out_hbm.at[pl.ds(obase + g * C, C)], ssem[b])
                bn = (b + LOOK) % NBUF
                gn = g + LOOK

                @pl.when(gn < nch)
                def _():
                    @pl.when(gn >= NBUF)
                    def _():
                        wait_store(bn)

                    issue_gather(gn, bn)

        for b in range(NBUF):
            wait_store(b)

    return gather


def kernel(edge_data, vertex_data, sender_idx, receiver_idx, W, b):
    B, E, D_EDGE = edge_data.shape
    _, N, D_FEAT = vertex_data.shape
    D_OUT = W.shape[1]

    edge2 = edge_data.reshape(E, D_EDGE)
    vert2 = vertex_data.reshape(N, D_FEAT)
    idx1 = sender_idx.reshape(E)
    We = W[:D_EDGE]
    Ws = W[D_EDGE : D_EDGE + D_FEAT]
    Wr = W[D_EDGE + D_FEAT :]
    b2 = b.reshape(1, D_OUT)

    vtab = pl.pallas_call(
        _vtab_body,
        out_shape=jax.ShapeDtypeStruct((N, D_OUT), jnp.float32),
    )(vert2, Ws, Wr, b2)

    K = 5
    ROWS = E // 128
    ROWS_PAD = _NW * 16 * K
    rows_k = ROWS_PAD // K
    Ek = rows_k * 128
    idx2d = jnp.pad(idx1, (0, ROWS_PAD * 128 - E)).reshape(ROWS_PAD, 128)
    Gs = [
        _make_gather(rows_k, D_OUT, roff=k * rows_k)(vtab, idx2d) for k in range(K)
    ]

    BE = 8192
    nblk = Ek // BE
    out = None
    for k in range(K):
        off = k * nblk
        in_specs = [
            pl.BlockSpec((BE, D_OUT), lambda i: (i, 0)),
            pl.BlockSpec((BE, D_EDGE), lambda i, off=off: (off + i, 0)),
            pl.BlockSpec((D_EDGE, D_OUT), lambda i: (0, 0)),
        ]
        args = [Gs[k], edge2, We]
        if out is None:
            body, kwargs = _combine_body, {}
        else:
            body, kwargs = _combine_alias_body, {"input_output_aliases": {0: 0}}
            in_specs = [pl.BlockSpec(memory_space=pl.ANY)] + in_specs
            args = [out] + args
        out = pl.pallas_call(
            body,
            grid=(nblk,),
            in_specs=in_specs,
            out_specs=pl.BlockSpec((BE, D_OUT), lambda i, off=off: (off + i, 0)),
            out_shape=jax.ShapeDtypeStruct((E, D_OUT), jnp.float32),
            **kwargs,
        )(*args)

    return out.reshape(B, E, D_OUT)


# speedup vs baseline: 1.9548x; 1.9548x over previous
import functools

import jax
import jax.numpy as jnp
from jax import lax
from jax.experimental import pallas as pl
from jax.experimental.pallas import tpu as pltpu
from jax.experimental.pallas import tpu_sc as plsc

_NC, _NS = 2, 16
_NW = _NC * _NS


def _vtab_body(v_ref, ws_ref, wr_ref, b_ref, o_ref):
    wsr = ws_ref[...] + wr_ref[...]
    o_ref[...] = (
        jnp.dot(v_ref[...], wsr, preferred_element_type=jnp.float32) + b_ref[...]
    )


def _combine_body(g_ref, e_ref, we_ref, o_ref):
    o_ref[...] = g_ref[...] + jnp.dot(
        e_ref[...].astype(jnp.bfloat16),
        we_ref[...].astype(jnp.bfloat16),
        preferred_element_type=jnp.float32,
    )


def _combine_alias_body(prev_ref, g_ref, e_ref, we_ref, o_ref):
    del prev_ref
    _combine_body(g_ref, e_ref, we_ref, o_ref)


def _make_gather(nch, C, D, koff):
    per_w = nch * C
    NBUF = 5
    LOOK = 2
    assert nch % NBUF == 0 and nch >= NBUF and C % 8 == 0 and C <= 128
    mesh = plsc.VectorSubcoreMesh(core_axis_name="c", subcore_axis_name="s")

    scratch = [pltpu.VMEM((per_w,), jnp.int32)]
    scratch += [pltpu.VMEM((C, D), jnp.float32) for _ in range(NBUF)]
    scratch += [pltpu.SemaphoreType.DMA for _ in range(2 * NBUF)]

    @functools.partial(
        pl.kernel,
        out_type=jax.ShapeDtypeStruct((per_w * _NW, D), jnp.float32),
        mesh=mesh,
        scratch_types=scratch,
    )
    def gather(vtab_hbm, idx_hbm, out_hbm, idx_v, *bufs_sems):
        rows = bufs_sems[:NBUF]
        gsem = bufs_sems[NBUF : 2 * NBUF]
        ssem = bufs_sems[2 * NBUF :]
        wid = lax.axis_index("s") * _NC + lax.axis_index("c")
        base = wid * per_w
        pltpu.sync_copy(idx_hbm.at[pl.ds(koff + base, per_w)], idx_v)

        def issue_gather(g, b):
            pltpu.async_copy(vtab_hbm.at[idx_v.at[pl.ds(g * C, C)]], rows[b], gsem[b])

        def wait_gather(g, b):
            pltpu.make_async_copy(
                vtab_hbm.at[idx_v.at[pl.ds(g * C, C)]], rows[b], gsem[b]
            ).wait()

        def wait_store(b):
            pltpu.make_async_copy(rows[b], out_hbm.at[pl.ds(base, C)], ssem[b]).wait()

        for p in range(LOOK):
            issue_gather(p, p)

        @pl.loop(0, nch, step=NBUF)
        def _(o):
            for b in range(NBUF):
                g = o + b
                wait_gather(g, b)
                pltpu.async_copy(rows[b], out_hbm.at[pl.ds(base + g * C, C)], ssem[b])
                bn = (b + LOOK) % NBUF
                gn = g + LOOK

                @pl.when(gn < nch)
                def _():
                    @pl.when(gn >= NBUF)
                    def _():
                        wait_store(bn)

                    issue_gather(gn, bn)

        for b in range(NBUF):
            wait_store(b)

    return gather


def kernel(edge_data, vertex_data, sender_idx, receiver_idx, W, b):
    B, E, D_EDGE = edge_data.shape
    _, N, D_FEAT = vertex_data.shape
    D_OUT = W.shape[1]

    edge2 = edge_data.reshape(E, D_EDGE)
    vert2 = vertex_data.reshape(N, D_FEAT)
    idx1 = sender_idx.reshape(E)
    We = W[:D_EDGE]
    Ws = W[D_EDGE : D_EDGE + D_FEAT]
    Wr = W[D_EDGE + D_FEAT :]
    b2 = b.reshape(1, D_OUT)

    vtab = pl.pallas_call(
        _vtab_body,
        out_shape=jax.ShapeDtypeStruct((N, D_OUT), jnp.float32),
    )(vert2, Ws, Wr, b2)

    C = 80
    SLICES = (10, 35, 35, 35, 10)
    assert sum(SLICES) * C * _NW == E
    BE = 6400

    Gs, starts = [], []
    koff = 0
    for nch in SLICES:
        starts.append(koff)
        Gs.append(_make_gather(nch, C, D_OUT, koff)(vtab, idx1))
        koff += nch * C * _NW

    out = None
    for k, nch in enumerate(SLICES):
        Ek = nch * C * _NW
        nblk = Ek // BE
        off = starts[k] // BE
        in_specs = [
            pl.BlockSpec((BE, D_OUT), lambda i: (i, 0)),
            pl.BlockSpec((BE, D_EDGE), lambda i, off=off: (off + i, 0)),
            pl.BlockSpec((D_EDGE, D_OUT), lambda i: (0, 0)),
        ]
        args = [Gs[k], edge2, We]
        if out is None:
            body, kwargs = _combine_body, {}
        else:
            body, kwargs = _combine_alias_body, {"input_output_aliases": {0: 0}}
            in_specs = [pl.BlockSpec(memory_space=pl.ANY)] + in_specs
            args = [out] + args
        out = pl.pallas_call(
            body,
            grid=(nblk,),
            in_specs=in_specs,
            out_specs=pl.BlockSpec((BE, D_OUT), lambda i, off=off: (off + i, 0)),
            out_shape=jax.ShapeDtypeStruct((E, D_OUT), jnp.float32),
            **kwargs,
        )(*args)

    return out.reshape(B, E, D_OUT)


# speedup vs baseline: 1.9562x; 1.0007x over previous
import functools

import jax
import jax.numpy as jnp
from jax import lax
from jax.experimental import pallas as pl
from jax.experimental.pallas import tpu as pltpu
from jax.experimental.pallas import tpu_sc as plsc

_NC, _NS = 2, 16
_NW = _NC * _NS


def _vtab_body(v_ref, ws_ref, wr_ref, b_ref, o_ref):
    wsr = (ws_ref[...] + wr_ref[...]).astype(jnp.bfloat16)
    o_ref[...] = (
        jnp.dot(
            v_ref[...].astype(jnp.bfloat16), wsr, preferred_element_type=jnp.float32
        )
        + b_ref[...]
    )


def _combine_body(g_ref, e_ref, we_ref, o_ref):
    o_ref[...] = g_ref[...] + jnp.dot(
        e_ref[...].astype(jnp.bfloat16),
        we_ref[...].astype(jnp.bfloat16),
        preferred_element_type=jnp.float32,
    )


def _combine_alias_body(prev_ref, g_ref, e_ref, we_ref, o_ref):
    del prev_ref
    _combine_body(g_ref, e_ref, we_ref, o_ref)


def _make_gather(nch, C, D, koff):
    per_w = nch * C
    NBUF = 5
    LOOK = 2
    assert nch % NBUF == 0 and nch >= NBUF and C % 8 == 0 and C <= 128
    mesh = plsc.VectorSubcoreMesh(core_axis_name="c", subcore_axis_name="s")

    scratch = [pltpu.VMEM((per_w,), jnp.int32)]
    scratch += [pltpu.VMEM((C, D), jnp.float32) for _ in range(NBUF)]
    scratch += [pltpu.SemaphoreType.DMA for _ in range(2 * NBUF)]

    @functools.partial(
        pl.kernel,
        out_type=jax.ShapeDtypeStruct((per_w * _NW, D), jnp.float32),
        mesh=mesh,
        scratch_types=scratch,
    )
    def gather(vtab_hbm, idx_hbm, out_hbm, idx_v, *bufs_sems):
        rows = bufs_sems[:NBUF]
        gsem = bufs_sems[NBUF : 2 * NBUF]
        ssem = bufs_sems[2 * NBUF :]
        wid = lax.axis_index("s") * _NC + lax.axis_index("c")
        base = wid * per_w
        pltpu.sync_copy(idx_hbm.at[pl.ds(koff + base, per_w)], idx_v)

        def issue_gather(g, b):
            pltpu.async_copy(vtab_hbm.at[idx_v.at[pl.ds(g * C, C)]], rows[b], gsem[b])

        def wait_gather(g, b):
            pltpu.make_async_copy(
                vtab_hbm.at[idx_v.at[pl.ds(g * C, C)]], rows[b], gsem[b]
            ).wait()

        def wait_store(b):
            pltpu.make_async_copy(rows[b], out_hbm.at[pl.ds(base, C)], ssem[b]).wait()

        for p in range(LOOK):
            issue_gather(p, p)

        @pl.loop(0, nch, step=NBUF)
        def _(o):
            for b in range(NBUF):
                g = o + b
                wait_gather(g, b)
                pltpu.async_copy(rows[b], out_hbm.at[pl.ds(base + g * C, C)], ssem[b])
                bn = (b + LOOK) % NBUF
                gn = g + LOOK

                @pl.when(gn < nch)
                def _():
                    @pl.when(gn >= NBUF)
                    def _():
                        wait_store(bn)

                    issue_gather(gn, bn)

        for b in range(NBUF):
            wait_store(b)

    return gather


def kernel(edge_data, vertex_data, sender_idx, receiver_idx, W, b):
    B, E, D_EDGE = edge_data.shape
    _, N, D_FEAT = vertex_data.shape
    D_OUT = W.shape[1]

    edge2 = edge_data.reshape(E, D_EDGE)
    vert2 = vertex_data.reshape(N, D_FEAT)
    idx1 = sender_idx.reshape(E)
    We = W[:D_EDGE]
    Ws = W[D_EDGE : D_EDGE + D_FEAT]
    Wr = W[D_EDGE + D_FEAT :]
    b2 = b.reshape(1, D_OUT)

    BV = 2000
    vtab = pl.pallas_call(
        _vtab_body,
        grid=(N // BV,),
        in_specs=[
            pl.BlockSpec((BV, D_FEAT), lambda i: (i, 0)),
            pl.BlockSpec((D_FEAT, D_OUT), lambda i: (0, 0)),
            pl.BlockSpec((D_FEAT, D_OUT), lambda i: (0, 0)),
            pl.BlockSpec((1, D_OUT), lambda i: (0, 0)),
        ],
        out_specs=pl.BlockSpec((BV, D_OUT), lambda i: (i, 0)),
        out_shape=jax.ShapeDtypeStruct((N, D_OUT), jnp.float32),
    )(vert2, Ws, Wr, b2)

    C = 80
    SLICES = (10, 35, 35, 35, 10)
    assert sum(SLICES) * C * _NW == E
    BE = 6400

    Gs, starts = [], []
    koff = 0
    for nch in SLICES:
        starts.append(koff)
        Gs.append(_make_gather(nch, C, D_OUT, koff)(vtab, idx1))
        koff += nch * C * _NW

    out = None
    for k, nch in enumerate(SLICES):
        Ek = nch * C * _NW
        nblk = Ek // BE
        off = starts[k] // BE
        in_specs = [
            pl.BlockSpec((BE, D_OUT), lambda i: (i, 0)),
            pl.BlockSpec((BE, D_EDGE), lambda i, off=off: (off + i, 0)),
            pl.BlockSpec((D_EDGE, D_OUT), lambda i: (0, 0)),
        ]
        args = [Gs[k], edge2, We]
        if out is None:
            body, kwargs = _combine_body, {}
        else:
            body, kwargs = _combine_alias_body, {"input_output_aliases": {0: 0}}
            in_specs = [pl.BlockSpec(memory_space=pl.ANY)] + in_specs
            args = [out] + args
        out = pl.pallas_call(
            body,
            grid=(nblk,),
            in_specs=in_specs,
            out_specs=pl.BlockSpec((BE, D_OUT), lambda i, off=off: (off + i, 0)),
            out_shape=jax.ShapeDtypeStruct((E, D_OUT), jnp.float32),
            **kwargs,
        )(*args)

    return out.reshape(B, E, D_OUT)


# speedup vs baseline: 1.9741x; 1.0091x over previous
import functools

import jax
import jax.numpy as jnp
from jax import lax
from jax.experimental import pallas as pl
from jax.experimental.pallas import tpu as pltpu
from jax.experimental.pallas import tpu_sc as plsc

_NC, _NS = 2, 16
_NW = _NC * _NS


def _vtab_body(v_ref, ws_ref, wr_ref, b_ref, o_ref):
    wsr = (ws_ref[...] + wr_ref[...]).astype(jnp.bfloat16)
    o_ref[...] = (
        jnp.dot(
            v_ref[...].astype(jnp.bfloat16), wsr, preferred_element_type=jnp.float32
        )
        + b_ref[...]
    )


def _combine_body(g_ref, e_ref, we_ref, o_ref):
    o_ref[...] = g_ref[...] + jnp.dot(
        e_ref[...].astype(jnp.bfloat16),
        we_ref[...].astype(jnp.bfloat16),
        preferred_element_type=jnp.float32,
    )


def _combine_alias_body(prev_ref, g_ref, e_ref, we_ref, o_ref):
    del prev_ref
    _combine_body(g_ref, e_ref, we_ref, o_ref)


def _make_gather(nch, C, D, koff):
    per_w = nch * C
    NBUF = 5
    LOOK = 2
    assert nch % NBUF == 0 and nch >= NBUF and C % 8 == 0 and C <= 128
    mesh = plsc.VectorSubcoreMesh(core_axis_name="c", subcore_axis_name="s")

    scratch = [pltpu.VMEM((per_w,), jnp.int32)]
    scratch += [pltpu.VMEM((C, D), jnp.float32) for _ in range(NBUF)]
    scratch += [pltpu.SemaphoreType.DMA for _ in range(2 * NBUF)]

    @functools.partial(
        pl.kernel,
        out_type=jax.ShapeDtypeStruct((per_w * _NW, D), jnp.float32),
        mesh=mesh,
        scratch_types=scratch,
    )
    def gather(vtab_hbm, idx_hbm, out_hbm, idx_v, *bufs_sems):
        rows = bufs_sems[:NBUF]
        gsem = bufs_sems[NBUF : 2 * NBUF]
        ssem = bufs_sems[2 * NBUF :]
        wid = lax.axis_index("s") * _NC + lax.axis_index("c")
        base = wid * per_w
        pltpu.sync_copy(idx_hbm.at[pl.ds(koff + base, per_w)], idx_v)

        def issue_gather(g, b):
            pltpu.async_copy(vtab_hbm.at[idx_v.at[pl.ds(g * C, C)]], rows[b], gsem[b])

        def wait_gather(g, b):
            pltpu.make_async_copy(
                vtab_hbm.at[idx_v.at[pl.ds(g * C, C)]], rows[b], gsem[b]
            ).wait()

        def wait_store(b):
            pltpu.make_async_copy(rows[b], out_hbm.at[pl.ds(base, C)], ssem[b]).wait()

        for p in range(LOOK):
            issue_gather(p, p)

        @pl.loop(0, nch, step=NBUF)
        def _(o):
            for b in range(NBUF):
                g = o + b
                wait_gather(g, b)
                pltpu.async_copy(rows[b], out_hbm.at[pl.ds(base + g * C, C)], ssem[b])
                bn = (b + LOOK) % NBUF
                gn = g + LOOK

                @pl.when(gn < nch)
                def _():
                    @pl.when(gn >= NBUF)
                    def _():
                        wait_store(bn)

                    issue_gather(gn, bn)

        for b in range(NBUF):
            wait_store(b)

    return gather


def kernel(edge_data, vertex_data, sender_idx, receiver_idx, W, b):
    B, E, D_EDGE = edge_data.shape
    _, N, D_FEAT = vertex_data.shape
    D_OUT = W.shape[1]

    edge2 = edge_data.reshape(E, D_EDGE)
    vert2 = vertex_data.reshape(N, D_FEAT)
    idx1 = sender_idx.reshape(E)
    We = W[:D_EDGE]
    Ws = W[D_EDGE : D_EDGE + D_FEAT]
    Wr = W[D_EDGE + D_FEAT :]
    b2 = b.reshape(1, D_OUT)

    BV = 2000
    vtab = pl.pallas_call(
        _vtab_body,
        grid=(N // BV,),
        in_specs=[
            pl.BlockSpec((BV, D_FEAT), lambda i: (i, 0)),
            pl.BlockSpec((D_FEAT, D_OUT), lambda i: (0, 0)),
            pl.BlockSpec((D_FEAT, D_OUT), lambda i: (0, 0)),
            pl.BlockSpec((1, D_OUT), lambda i: (0, 0)),
        ],
        out_specs=pl.BlockSpec((BV, D_OUT), lambda i: (i, 0)),
        out_shape=jax.ShapeDtypeStruct((N, D_OUT), jnp.float32),
    )(vert2, Ws, Wr, b2)

    C = 80
    SLICES = (25, 25, 25, 25, 25)
    assert sum(SLICES) * C * _NW == E
    BE = 6400

    Gs, starts = [], []
    koff = 0
    for nch in SLICES:
        starts.append(koff)
        Gs.append(_make_gather(nch, C, D_OUT, koff)(vtab, idx1))
        koff += nch * C * _NW

    out = None
    for k, nch in enumerate(SLICES):
        Ek = nch * C * _NW
        nblk = Ek // BE
        off = starts[k] // BE
        in_specs = [
            pl.BlockSpec((BE, D_OUT), lambda i: (i, 0)),
            pl.BlockSpec((BE, D_EDGE), lambda i, off=off: (off + i, 0)),
            pl.BlockSpec((D_EDGE, D_OUT), lambda i: (0, 0)),
        ]
        args = [Gs[k], edge2, We]
        if out is None:
            body, kwargs = _combine_body, {}
        else:
            body, kwargs = _combine_alias_body, {"input_output_aliases": {0: 0}}
            in_specs = [pl.BlockSpec(memory_space=pl.ANY)] + in_specs
            args = [out] + args
        out = pl.pallas_call(
            body,
            grid=(nblk,),
            in_specs=in_specs,
            out_specs=pl.BlockSpec((BE, D_OUT), lambda i, off=off: (off + i, 0)),
            out_shape=jax.ShapeDtypeStruct((E, D_OUT), jnp.float32),
            **kwargs,
        )(*args)

    return out.reshape(B, E, D_OUT)
